# Initial kernel scaffold; baseline (speedup 1.0000x reference)
import jax
import jax.numpy as jnp
from jax.experimental import pallas as pl


def kernel(x, W_in, b_in, embed, W_out, b_out):
    raise NotImplementedError("write your pallas kernel here")


# speedup vs baseline: 1.5033x; 1.5033x over previous
import functools

import jax
import jax.numpy as jnp
from jax import lax
from jax.experimental import pallas as pl
from jax.experimental.pallas import tpu as pltpu
from jax.experimental.pallas import tpu_sc as plsc

BSZ, T, C = 4, 4096, 1024
D = 1024
K = 2048
DS = 2
N = T // DS
M = BSZ * N

BM = 256
BK = 512
CH = 64


def _lut_body(embed_ref, wout_ref, bout_ref, lut_ref, en_ref):
    e = embed_ref[:]
    lut_ref[:] = lax.dot_general(
        e, wout_ref[:], (((1,), (1,)), ((), ())),
        preferred_element_type=jnp.float32) + bout_ref[:]
    en_ref[:] = jnp.sum(e * e, axis=1)


def _dist_body(xw_ref, wf_ref, bin_ref, embed_ref, en_ref,
               idx_ref, loss_ref, acc_ref):
    i = pl.program_id(0)
    z = lax.dot_general(
        xw_ref[:], wf_ref[:], (((1,), (1,)), ((), ())),
        preferred_element_type=jnp.float32) + bin_ref[:]
    s = lax.dot_general(
        z, embed_ref[:], (((1,), (1,)), ((), ())),
        preferred_element_type=jnp.float32)
    zn = jnp.sum(z * z, axis=1, keepdims=True)
    dist = (zn - 2.0 * s) + en_ref[:][None, :]
    idx_ref[:] = jnp.argmin(dist, axis=1).astype(jnp.int32)
    mind = jnp.min(dist, axis=1)

    @pl.when(i == 0)
    def _():
        acc_ref[0] = 0.0

    acc_ref[0] += jnp.sum(mind)

    @pl.when(i == pl.num_programs(0) - 1)
    def _():
        loss_ref[0, 0] = acc_ref[0] / (M * D)


NC = 2
NS = 16
NW = NC * NS


@functools.lru_cache(maxsize=1)
def _make_gather():
    ids_per_w = M // NW
    mesh = plsc.VectorSubcoreMesh(core_axis_name="c", subcore_axis_name="s")
    nc = NC

    @functools.partial(
        pl.kernel, mesh=mesh,
        out_type=jax.ShapeDtypeStruct((M, DS, C), jnp.float32),
        scratch_types=[
            pltpu.VMEM((CH,), jnp.int32),
            pltpu.VMEM((CH, C), jnp.float32),
            pltpu.SemaphoreType.DMA,
        ],
    )
    def gather(lut_hbm, idx_hbm, out_hbm, idx_v, rows_v, sem):
        wid = lax.axis_index("s") * nc + lax.axis_index("c")
        base = wid * ids_per_w

        def chunk(j, carry):
            off = base + j * CH
            pltpu.sync_copy(idx_hbm.at[pl.ds(off, CH)], idx_v)
            pltpu.async_copy(lut_hbm.at[idx_v], rows_v, sem).wait()
            pltpu.sync_copy(rows_v, out_hbm.at[pl.ds(off, CH), 0, :])
            pltpu.sync_copy(rows_v, out_hbm.at[pl.ds(off, CH), 1, :])
            return carry

        lax.fori_loop(0, ids_per_w // CH, chunk, 0)

    return gather


def kernel(x, W_in, b_in, embed, W_out, b_out):
    xw = x.reshape(M, DS * C)
    wf = W_in.transpose(0, 2, 1).reshape(D, DS * C)
    wout = W_out[:, :, 0]

    lut, en = pl.pallas_call(
        _lut_body,
        grid=(K // BK,),
        in_specs=[
            pl.BlockSpec((BK, D), lambda i: (i, 0)),
            pl.BlockSpec((C, D), lambda i: (0, 0)),
            pl.BlockSpec((1, C), lambda i: (0, 0)),
        ],
        out_specs=[
            pl.BlockSpec((BK, C), lambda i: (i, 0)),
            pl.BlockSpec((BK,), lambda i: (i,)),
        ],
        out_shape=[
            jax.ShapeDtypeStruct((K, C), jnp.float32),
            jax.ShapeDtypeStruct((K,), jnp.float32),
        ],
    )(embed, wout, b_out.reshape(1, C))

    idx, loss = pl.pallas_call(
        _dist_body,
        grid=(M // BM,),
        in_specs=[
            pl.BlockSpec((BM, DS * C), lambda i: (i, 0)),
            pl.BlockSpec((D, DS * C), lambda i: (0, 0)),
            pl.BlockSpec((1, D), lambda i: (0, 0)),
            pl.BlockSpec((K, D), lambda i: (0, 0)),
            pl.BlockSpec((K,), lambda i: (0,)),
        ],
        out_specs=[
            pl.BlockSpec((BM,), lambda i: (i,)),
            pl.BlockSpec((1, 1), lambda i: (0, 0),
                         memory_space=pltpu.SMEM),
        ],
        out_shape=[
            jax.ShapeDtypeStruct((M,), jnp.int32),
            jax.ShapeDtypeStruct((1, 1), jnp.float32),
        ],
        scratch_shapes=[pltpu.SMEM((1,), jnp.float32)],
    )(xw, wf, b_in.reshape(1, D), embed, en)

    out = _make_gather()(lut, idx)
    return (out.reshape(BSZ, T, C), loss.reshape(()))
